# Initial kernel scaffold; baseline (speedup 1.0000x reference)
import jax
import jax.numpy as jnp
from jax.experimental import pallas as pl


def kernel(feats, edge_dict, Wq, bq, Wk, bk, Wv, bv):
    raise NotImplementedError("write your pallas kernel here")


# speedup vs baseline: 3.3401x; 3.3401x over previous
import functools

import jax
import jax.numpy as jnp
from jax import lax
from jax.experimental import pallas as pl
from jax.experimental.pallas import tpu as pltpu
from jax.experimental.pallas import tpu_sc as plsc

_NC, _NS, _L = 2, 16, 16
_NW = _NC * _NS


def _proj_body(w_ref, f_ref, b_ref, o_ref):
    o_ref[...] = (
        lax.dot_general(
            w_ref[...],
            f_ref[...],
            dimension_numbers=(((1,), (1,)), ((), ())),
            preferred_element_type=jnp.float32,
        )
        + b_ref[...]
    )


@functools.lru_cache(maxsize=None)
def _make_project(N, D):
    return pl.pallas_call(
        _proj_body,
        out_shape=jax.ShapeDtypeStruct((16, N), jnp.float32),
    )


@functools.lru_cache(maxsize=None)
def _make_sc_attend(N, E, K, D):
    EPG = _L
    EPW = E // _NW
    G = EPW // EPG
    R = EPG * K
    CH = D // _L
    mesh = plsc.VectorSubcoreMesh(core_axis_name="c", subcore_axis_name="s")

    @functools.partial(
        pl.kernel,
        out_type=jax.ShapeDtypeStruct((E, D), jnp.float32),
        mesh=mesh,
        compiler_params=pltpu.CompilerParams(needs_layout_passes=False),
        scratch_types=[
            pltpu.VMEM((3, N), jnp.float32),
            pltpu.VMEM((R,), jnp.int32),
            pltpu.VMEM((R, D), jnp.float32),
            pltpu.VMEM((EPG, D), jnp.float32),
            pltpu.VMEM((R,), jnp.float32),
            pltpu.SemaphoreType.DMA,
        ],
    )
    def sc_attend(feats_hbm, ed_hbm, p_hbm, out_hbm,
                  qkv_v, idx_v, rows_v, out_v, d_v, sem_r):
        wid = lax.axis_index("s") * _NC + lax.axis_index("c")
        ebase = wid * EPW
        pltpu.sync_copy(p_hbm.at[pl.ds(0, 3)], qkv_v)

        @pl.loop(0, G)
        def _group(g):
            e0 = ebase + g * EPG
            pltpu.sync_copy(ed_hbm.at[pl.ds(e0 * K, R)], idx_v)
            cp_r = pltpu.async_copy(feats_hbm.at[idx_v], rows_v, sem_r)

            ii = lax.iota(jnp.int32, _L)
            vid = [plsc.load_gather(idx_v, [ii * K + j]) for j in range(K)]
            row = [jnp.full((_L,), r, jnp.int32) for r in range(3)]
            q = [plsc.load_gather(qkv_v, [row[0], vid[j]]) for j in range(K)]
            k = [plsc.load_gather(qkv_v, [row[1], vid[j]]) for j in range(K)]
            v = [plsc.load_gather(qkv_v, [row[2], vid[j]]) for j in range(K)]
            for j in range(K):
                logits = [q[j] * k[m] for m in range(K)]
                ms = [m for m in range(K) if m != j]
                mx = logits[ms[0]]
                for m in ms[1:]:
                    mx = jnp.maximum(mx, logits[m])
                s = None
                num = None
                for m in ms:
                    ex = jnp.exp(logits[m] - mx)
                    s = ex if s is None else s + ex
                    w = ex * v[m]
                    num = w if num is None else num + w
                r = num / s
                d_v[pl.ds(j * _L, _L)] = 1.0 - 2.0 / (jnp.exp(r + r) + 1.0)

            cp_r.wait()

            @pl.loop(0, EPG)
            def _edge(i):
                db = [
                    plsc.load_gather(
                        d_v, [jnp.full((_L,), j * _L, jnp.int32) + i]
                    )
                    for j in range(K)
                ]

                @pl.loop(0, CH)
                def _chunk(c):
                    acc = db[0] * rows_v[i * K, pl.ds(c * _L, _L)]
                    for j in range(1, K):
                        acc = acc + db[j] * rows_v[i * K + j, pl.ds(c * _L, _L)]
                    out_v[i, pl.ds(c * _L, _L)] = acc

            pltpu.sync_copy(out_v, out_hbm.at[pl.ds(e0, EPG)])

    return sc_attend


def kernel(feats, edge_dict, Wq, bq, Wk, bk, Wv, bv):
    N, D = feats.shape
    E, K = edge_dict.shape
    wcat = jnp.zeros((16, D), jnp.float32)
    wcat = wcat.at[0, :].set(Wq[0]).at[1, :].set(Wk[0]).at[2, :].set(Wv[0])
    bcat = jnp.zeros((16, 1), jnp.float32)
    bcat = bcat.at[0, 0].set(bq[0]).at[1, 0].set(bk[0]).at[2, 0].set(bv[0])
    packed = _make_project(N, D)(wcat, feats, bcat)
    return _make_sc_attend(N, E, K, D)(feats, edge_dict.reshape(-1), packed)


# speedup vs baseline: 3.9199x; 1.1736x over previous
import functools

import jax
import jax.numpy as jnp
from jax import lax
from jax.experimental import pallas as pl
from jax.experimental.pallas import tpu as pltpu
from jax.experimental.pallas import tpu_sc as plsc

_NC, _NS, _L = 2, 16, 16
_NW = _NC * _NS


def _proj_body(wq, wk, wv, bq, bk, bv, f, o):
    w3 = jnp.concatenate([wq[...], wk[...], wv[...]], axis=0)
    b3 = jnp.concatenate([bq[...], bk[...], bv[...]], axis=0)[:, None]
    o[...] = (
        lax.dot_general(
            w3,
            f[...],
            dimension_numbers=(((1,), (1,)), ((), ())),
            preferred_element_type=jnp.float32,
        )
        + b3
    )


@functools.lru_cache(maxsize=None)
def _make_project(N, D):
    return pl.pallas_call(
        _proj_body,
        out_shape=jax.ShapeDtypeStruct((3, N), jnp.float32),
    )


@functools.lru_cache(maxsize=None)
def _make_sc_attend(N, E, K, D):
    EPG = _L
    EPW = E // _NW
    G = EPW // EPG
    R = EPG * K
    CH = D // _L
    mesh = plsc.VectorSubcoreMesh(core_axis_name="c", subcore_axis_name="s")

    @functools.partial(
        pl.kernel,
        out_type=jax.ShapeDtypeStruct((E, D), jnp.float32),
        mesh=mesh,
        compiler_params=pltpu.CompilerParams(needs_layout_passes=False),
        scratch_types=[
            pltpu.VMEM((3, N), jnp.float32),
            pltpu.VMEM((R,), jnp.int32),
            pltpu.VMEM((R,), jnp.int32),
            pltpu.VMEM((R, D), jnp.float32),
            pltpu.VMEM((R, D), jnp.float32),
            pltpu.VMEM((EPG, D), jnp.float32),
            pltpu.VMEM((R,), jnp.float32),
            pltpu.SemaphoreType.DMA,
            pltpu.SemaphoreType.DMA,
        ],
    )
    def sc_attend(feats_hbm, ed_hbm, p_hbm, out_hbm,
                  qkv_v, idx0_v, idx1_v, rows0_v, rows1_v, out_v, d_v,
                  sem0, sem1):
        wid = lax.axis_index("s") * _NC + lax.axis_index("c")
        ebase = wid * EPW
        pltpu.sync_copy(p_hbm, qkv_v)

        def fire(g, idx_v, rows_v, sem):
            e0 = ebase + g * EPG
            pltpu.sync_copy(ed_hbm.at[pl.ds(e0 * K, R)], idx_v)
            pltpu.async_copy(feats_hbm.at[idx_v], rows_v, sem)

        def wait(idx_v, rows_v, sem):
            pltpu.make_async_copy(feats_hbm.at[idx_v], rows_v, sem).wait()

        def compute(g, idx_v, rows_v):
            ii = lax.iota(jnp.int32, _L)
            vid = [plsc.load_gather(idx_v, [ii * K + j]) for j in range(K)]
            row = [jnp.full((_L,), r, jnp.int32) for r in range(3)]
            q = [plsc.load_gather(qkv_v, [row[0], vid[j]]) for j in range(K)]
            k = [plsc.load_gather(qkv_v, [row[1], vid[j]]) for j in range(K)]
            v = [plsc.load_gather(qkv_v, [row[2], vid[j]]) for j in range(K)]
            for j in range(K):
                logits = [q[j] * k[m] for m in range(K)]
                ms = [m for m in range(K) if m != j]
                mx = logits[ms[0]]
                for m in ms[1:]:
                    mx = jnp.maximum(mx, logits[m])
                s = None
                num = None
                for m in ms:
                    ex = jnp.exp(logits[m] - mx)
                    s = ex if s is None else s + ex
                    w = ex * v[m]
                    num = w if num is None else num + w
                r = num / s
                d_v[pl.ds(j * _L, _L)] = 1.0 - 2.0 / (jnp.exp(r + r) + 1.0)

            @pl.loop(0, EPG)
            def _edge(i):
                db = [
                    plsc.load_gather(
                        d_v, [jnp.full((_L,), j * _L, jnp.int32) + i]
                    )
                    for j in range(K)
                ]

                @pl.loop(0, CH)
                def _chunk(c):
                    acc = db[0] * rows_v[i * K, pl.ds(c * _L, _L)]
                    for j in range(1, K):
                        acc = acc + db[j] * rows_v[i * K + j, pl.ds(c * _L, _L)]
                    out_v[i, pl.ds(c * _L, _L)] = acc

            pltpu.sync_copy(out_v, out_hbm.at[pl.ds(ebase + g * EPG, EPG)])

        fire(0, idx0_v, rows0_v, sem0)
        fire(1, idx1_v, rows1_v, sem1)

        @pl.loop(0, G, step=2)
        def _group(g):
            wait(idx0_v, rows0_v, sem0)
            compute(g, idx0_v, rows0_v)

            @pl.when(g + 2 < G)
            def _():
                fire(g + 2, idx0_v, rows0_v, sem0)

            wait(idx1_v, rows1_v, sem1)
            compute(g + 1, idx1_v, rows1_v)

            @pl.when(g + 3 < G)
            def _():
                fire(g + 3, idx1_v, rows1_v, sem1)

    return sc_attend


def kernel(feats, edge_dict, Wq, bq, Wk, bk, Wv, bv):
    N, D = feats.shape
    E, K = edge_dict.shape
    qkv = _make_project(N, D)(Wq, Wk, Wv, bq, bk, bv, feats)
    return _make_sc_attend(N, E, K, D)(feats, edge_dict.reshape(-1), qkv)


# speedup vs baseline: 5.2942x; 1.3506x over previous
import functools

import jax
import jax.numpy as jnp
from jax import lax
from jax.experimental import pallas as pl
from jax.experimental.pallas import tpu as pltpu
from jax.experimental.pallas import tpu_sc as plsc

_NC, _NS, _L = 2, 16, 16
_NW = _NC * _NS


def _proj_body(wq, wk, wv, bq, bk, bv, f, o):
    w3 = jnp.concatenate([wq[...], wk[...], wv[...]], axis=0)
    b3 = jnp.concatenate([bq[...], bk[...], bv[...]], axis=0)[:, None]
    o[...] = (
        lax.dot_general(
            w3,
            f[...],
            dimension_numbers=(((1,), (1,)), ((), ())),
            preferred_element_type=jnp.float32,
        )
        + b3
    )


@functools.lru_cache(maxsize=None)
def _make_project(N, D):
    return pl.pallas_call(
        _proj_body,
        out_shape=jax.ShapeDtypeStruct((3, N), jnp.float32),
    )


@functools.lru_cache(maxsize=None)
def _make_sc_attend(N, E, K, D):
    EPG = _L
    EPW = E // _NW
    G = EPW // EPG
    R = EPG * K
    CH = D // _L
    mesh = plsc.VectorSubcoreMesh(core_axis_name="c", subcore_axis_name="s")

    @functools.partial(
        pl.kernel,
        out_type=jax.ShapeDtypeStruct((E, D), jnp.float32),
        mesh=mesh,
        compiler_params=pltpu.CompilerParams(needs_layout_passes=False),
        scratch_types=[
            pltpu.VMEM((3, N), jnp.float32),
            pltpu.VMEM((EPW * K,), jnp.int32),
            pltpu.VMEM((R, D), jnp.float32),
            pltpu.VMEM((R, D), jnp.float32),
            pltpu.VMEM((EPG, D), jnp.float32),
            pltpu.VMEM((EPG, D), jnp.float32),
            pltpu.VMEM((R,), jnp.float32),
            pltpu.SemaphoreType.DMA,
            pltpu.SemaphoreType.DMA,
            pltpu.SemaphoreType.DMA,
            pltpu.SemaphoreType.DMA,
        ],
    )
    def sc_attend(feats_hbm, ed_hbm, p_hbm, out_hbm,
                  qkv_v, aidx_v, rows0_v, rows1_v, out0_v, out1_v, d_v,
                  sem_r0, sem_r1, sem_o0, sem_o1):
        wid = lax.axis_index("s") * _NC + lax.axis_index("c")
        ebase = wid * EPW
        pltpu.sync_copy(ed_hbm.at[pl.ds(ebase * K, EPW * K)], aidx_v)
        pltpu.sync_copy(p_hbm, qkv_v)

        def fire(g, rows_v, sem):
            pltpu.async_copy(
                feats_hbm.at[aidx_v.at[pl.ds(g * R, R)]], rows_v, sem
            )

        def wait_rows(g, rows_v, sem):
            pltpu.make_async_copy(
                feats_hbm.at[aidx_v.at[pl.ds(g * R, R)]], rows_v, sem
            ).wait()

        def out_ref(g):
            return out_hbm.at[pl.ds(ebase + g * EPG, EPG)]

        def compute(g, rows_v, out_v, sem_o):
            ii = lax.iota(jnp.int32, _L)
            base = g * R
            vid = [
                plsc.load_gather(aidx_v, [ii * K + (base + j)])
                for j in range(K)
            ]
            row = [jnp.full((_L,), r, jnp.int32) for r in range(3)]
            q = [plsc.load_gather(qkv_v, [row[0], vid[j]]) for j in range(K)]
            k = [plsc.load_gather(qkv_v, [row[1], vid[j]]) for j in range(K)]
            v = [plsc.load_gather(qkv_v, [row[2], vid[j]]) for j in range(K)]
            for j in range(K):
                logits = [q[j] * k[m] for m in range(K)]
                ms = [m for m in range(K) if m != j]
                mx = logits[ms[0]]
                for m in ms[1:]:
                    mx = jnp.maximum(mx, logits[m])
                s = None
                num = None
                for m in ms:
                    ex = jnp.exp(logits[m] - mx)
                    s = ex if s is None else s + ex
                    w = ex * v[m]
                    num = w if num is None else num + w
                r = num / s
                d_v[pl.ds(j * _L, _L)] = 1.0 - 2.0 / (jnp.exp(r + r) + 1.0)

            @pl.when(g >= 2)
            def _():
                pltpu.make_async_copy(out_v, out_ref(g - 2), sem_o).wait()

            @plsc.parallel_loop(0, EPG)
            def _edge(i):
                db = [
                    plsc.load_gather(
                        d_v, [jnp.full((_L,), j * _L, jnp.int32) + i]
                    )
                    for j in range(K)
                ]

                @plsc.parallel_loop(0, CH, unroll=4)
                def _chunk(c):
                    acc = db[0] * rows_v[i * K, pl.ds(c * _L, _L)]
                    for j in range(1, K):
                        acc = acc + db[j] * rows_v[i * K + j, pl.ds(c * _L, _L)]
                    out_v[i, pl.ds(c * _L, _L)] = acc

            pltpu.async_copy(out_v, out_ref(g), sem_o)

        fire(0, rows0_v, sem_r0)
        fire(1, rows1_v, sem_r1)

        @pl.loop(0, G, step=2)
        def _group(g):
            wait_rows(g, rows0_v, sem_r0)
            compute(g, rows0_v, out0_v, sem_o0)

            @pl.when(g + 2 < G)
            def _():
                fire(g + 2, rows0_v, sem_r0)

            wait_rows(g + 1, rows1_v, sem_r1)
            compute(g + 1, rows1_v, out1_v, sem_o1)

            @pl.when(g + 3 < G)
            def _():
                fire(g + 3, rows1_v, sem_r1)

        pltpu.make_async_copy(out0_v, out_ref(G - 2), sem_o0).wait()
        pltpu.make_async_copy(out1_v, out_ref(G - 1), sem_o1).wait()

    return sc_attend


def kernel(feats, edge_dict, Wq, bq, Wk, bk, Wv, bv):
    N, D = feats.shape
    E, K = edge_dict.shape
    qkv = _make_project(N, D)(Wq, Wk, Wv, bq, bk, bv, feats)
    return _make_sc_attend(N, E, K, D)(feats, edge_dict.reshape(-1), qkv)
